# baseline (speedup 1.0000x reference)
import jax
import jax.numpy as jnp
from jax.experimental import pallas as pl
from jax.experimental.pallas import tpu as pltpu

PC = (-51.2, -51.2, -5.0, 51.2, 51.2, 3.0)
E = 512
NH = 8
DH = 64
TOPK = 512
DS = 4
C = 256
HP = 64
WP = 64
T = HP * WP
N = 1220


def _pool_body(pts_ref, aw_ref, bev_ref, score_ref):
    x = pts_ref[0]
    aw = aw_ref[...]
    dims = (((1,), (1,)), ((), ()))
    for i in range(8):
        z = jnp.sum(x[:, 4 * i:4 * i + 4, :], axis=1)
        zh = z.astype(jnp.bfloat16)
        zl = (z - zh.astype(jnp.float32)).astype(jnp.bfloat16)
        z2 = jnp.concatenate([zh, zl], axis=1)
        pooled = jax.lax.dot_general(aw, z2, dims,
                                     preferred_element_type=jnp.float32)
        bev_ref[0, WP * i:WP * (i + 1), :] = pooled
        score_ref[0, WP * i:WP * (i + 1), :] = (
            jnp.sum(pooled * pooled, axis=1, keepdims=True) * (1.0 / C))


def _ln(x, g, b):
    m = jnp.mean(x, axis=1, keepdims=True)
    v = jnp.mean((x - m) * (x - m), axis=1, keepdims=True)
    return (x - m) * jax.lax.rsqrt(v + 1e-5) * g + b


def _mm_t(x, w):
    return jax.lax.dot_general(x, w, (((1,), (1,)), ((), ())),
                               preferred_element_type=jnp.float32)


def _mm_n(x, w):
    return jnp.dot(x, w, preferred_element_type=jnp.float32)


def _fused_body(score_ref, bev_ref, inst_ref, anc_ref,
                qw_ref, kw_ref, vw_ref, bq_ref, bk_ref, bv_ref,
                aow_ref, aob_ref, ow_ref, ob_ref,
                qng_ref, qnb_ref, tng_ref, tnb_ref, ong_ref, onb_ref,
                posw1_ref, posb1_ref, posw2_ref, posb2_ref,
                ancw1_ref, ancb1_ref, ancw2_ref, ancb2_ref,
                gw1_ref, gb1_ref, gw2_ref, gb2_ref,
                out_ref):
    s = score_ref[0]
    bits = jax.lax.bitcast_convert_type(s, jnp.int32)
    lane_t = jax.lax.broadcasted_iota(jnp.int32, (1, T), 1)

    def count_ge(t):
        return jnp.sum((bits >= t).astype(jnp.int32))

    def bs_body(_, lohi):
        lo, hi = lohi
        span = hi - lo
        mid = lo + (span >> 1) + (span & 1)
        ok = count_ge(mid) >= TOPK
        return (jnp.where(ok, mid, lo), jnp.where(ok, hi, mid - 1))

    lo0 = jnp.int32(0)
    hi0 = jnp.int32(0x7F800000)
    V, _ = jax.lax.fori_loop(0, 31, bs_body, (lo0, hi0))

    n_hi = count_ge(V + 1)
    m_tie = TOPK - n_hi
    tie = bits == V

    def bs2_body(_, lohi):
        lo, hi = lohi
        mid = (lo + hi) >> 1
        cnt = jnp.sum((tie & (lane_t < mid)).astype(jnp.int32))
        ok = cnt >= m_tie
        return (jnp.where(ok, lo, mid + 1), jnp.where(ok, mid, hi))

    _, j_tie = jax.lax.fori_loop(0, 13, bs2_body,
                                 (jnp.int32(0), jnp.int32(T)))

    sel = (bits > V) | (tie & (lane_t < j_tie))
    sel_i = sel.astype(jnp.int32)

    c = sel_i
    k = 1
    while k < T:
        c = c + jnp.where(lane_t >= k, pltpu.roll(c, k, axis=1), 0)
        k *= 2
    rank = c - sel_i

    row_id = jax.lax.broadcasted_iota(jnp.int32, (TOPK, T), 0)
    onehot = (sel & (rank == row_id)).astype(jnp.float32)

    bev = bev_ref[0]
    lidar = jnp.dot(onehot, bev, preferred_element_type=jnp.float32)

    xs_n = ((lane_t % WP).astype(jnp.float32) + 0.5) * (1.0 / WP)
    ys_n = ((lane_t // WP).astype(jnp.float32) + 0.5) * (1.0 / HP)
    dims_t = (((1,), (1,)), ((), ()))
    xsel = jax.lax.dot_general(onehot, xs_n, dims_t,
                               preferred_element_type=jnp.float32)
    ysel = jax.lax.dot_general(onehot, ys_n, dims_t,
                               preferred_element_type=jnp.float32)
    w1 = posw1_ref[...]
    h1 = jnp.maximum(xsel * w1[0:1, :] + ysel * w1[1:2, :] + posb1_ref[...],
                     0.0)
    lidar = lidar + _mm_t(h1, posw2_ref[...]) + posb2_ref[...]

    kvn = _ln(lidar, tng_ref[...], tnb_ref[...])

    xy = anc_ref[0]
    xn = jnp.clip((xy[:, 0:1] - PC[0]) / (PC[3] - PC[0]), 0.0, 1.0)
    yn = jnp.clip((xy[:, 1:2] - PC[1]) / (PC[4] - PC[1]), 0.0, 1.0)
    aw1 = ancw1_ref[...]
    a1 = jnp.maximum(xn * aw1[0:1, :] + yn * aw1[1:2, :] + ancb1_ref[...],
                     0.0)
    query_input = inst_ref[0] + _mm_t(a1, ancw2_ref[...]) + ancb2_ref[...]
    query = _ln(query_input, qng_ref[...], qnb_ref[...])

    qw = qw_ref[...]
    kw = kw_ref[...]
    vw = vw_ref[...]
    aow = aow_ref[...]
    bq = bq_ref[...]
    bk = bk_ref[...]
    bv = bv_ref[...]
    scale = 1.0 / (DH ** 0.5)
    o1 = jnp.zeros((N, E), jnp.float32)
    for h in range(NH):
        sl = slice(h * DH, (h + 1) * DH)
        qh = (_mm_t(query, qw[sl, :]) + bq[h]) * scale
        kh = _mm_t(kvn, kw[sl, :]) + bk[h]
        vh = _mm_t(kvn, vw[sl, :]) + bv[h]
        logits = _mm_t(qh, kh)
        mx = jnp.max(logits, axis=1, keepdims=True)
        e = jnp.exp(logits - mx)
        p = e / jnp.sum(e, axis=1, keepdims=True)
        ctxh = _mm_n(p, vh)
        o1 = o1 + _mm_n(ctxh, aow[sl, :])
    o1 = o1 + aob_ref[...]
    o2 = _mm_t(o1, ow_ref[...]) + ob_ref[...]
    o3 = _ln(o2, ong_ref[...], onb_ref[...])

    g1 = jnp.maximum(_mm_t(query_input, gw1_ref[...]) + gb1_ref[...], 0.0)
    g2 = _mm_t(g1, gw2_ref[...]) + gb2_ref[...]
    gate = 1.0 / (1.0 + jnp.exp(-g2))
    out_ref[0] = inst_ref[0] + gate * o3


def kernel(instance_feature, pts_feats, anchor, q_proj_w, k_proj_w, v_proj_w,
           in_proj_b, attn_out_w, attn_out_b, qn_g, qn_b, tn_g, tn_b, on_g,
           on_b, out_w, out_b, pos_w1, pos_b1, pos_w2, pos_b2, anc_w1,
           anc_b1, anc_w2, anc_b2, gate_w1, gate_b1, gate_w2, gate_b2):
    B = pts_feats.shape[0]
    W = pts_feats.shape[3]

    aw = (jnp.equal(jax.lax.broadcasted_iota(jnp.int32, (WP, W), 0),
                    jax.lax.broadcasted_iota(jnp.int32, (WP, W), 1) // DS)
          .astype(jnp.bfloat16) * (1.0 / (DS * DS)))
    aw = jnp.concatenate([aw, aw], axis=1)

    bev, score = pl.pallas_call(
        _pool_body,
        grid=(B, HP // 8),
        in_specs=[
            pl.BlockSpec((1, C, 8 * DS, W), lambda b, h: (b, 0, h, 0)),
            pl.BlockSpec((WP, 2 * W), lambda b, h: (0, 0)),
        ],
        out_specs=[
            pl.BlockSpec((1, 8 * WP, C), lambda b, h: (b, h, 0)),
            pl.BlockSpec((1, 8 * WP, 1), lambda b, h: (b, h, 0)),
        ],
        out_shape=[
            jax.ShapeDtypeStruct((B, T, C), jnp.float32),
            jax.ShapeDtypeStruct((B, T, 1), jnp.float32),
        ],
    )(pts_feats, aw)

    score = score.reshape(B, 1, T)
    row = lambda x: x.reshape(1, -1)
    full = lambda *shape: pl.BlockSpec(shape, lambda b: (0,) * len(shape))
    perb = lambda *shape: pl.BlockSpec((1,) + shape,
                                       lambda b: (b,) + (0,) * len(shape))
    bq = in_proj_b[0:E].reshape(NH, 1, DH)
    bk = in_proj_b[E:2 * E].reshape(NH, 1, DH)
    bv = in_proj_b[2 * E:].reshape(NH, 1, DH)

    out = pl.pallas_call(
        _fused_body,
        grid=(B,),
        in_specs=[
            perb(1, T), perb(T, C), perb(N, E), perb(N, 2),
            full(E, E), full(E, C), full(E, C),
            full(NH, 1, DH), full(NH, 1, DH), full(NH, 1, DH),
            full(E, E), full(1, E), full(E, E), full(1, E),
            full(1, E), full(1, E), full(1, C), full(1, C),
            full(1, E), full(1, E),
            full(2, C), full(1, C), full(C, C), full(1, C),
            full(2, E), full(1, E), full(E, E), full(1, E),
            full(E, E), full(1, E), full(E, E), full(1, E),
        ],
        out_specs=pl.BlockSpec((1, N, E), lambda b: (b, 0, 0)),
        out_shape=jax.ShapeDtypeStruct((B, N, E), jnp.float32),
    )(score, bev, instance_feature, anchor[..., 0:2],
      q_proj_w, k_proj_w, v_proj_w, bq, bk, bv,
      attn_out_w.T, row(attn_out_b), out_w, row(out_b),
      row(qn_g), row(qn_b), row(tn_g), row(tn_b), row(on_g), row(on_b),
      pos_w1.T, row(pos_b1), pos_w2, row(pos_b2),
      anc_w1.T, row(anc_b1), anc_w2, row(anc_b2),
      gate_w1, row(gate_b1), gate_w2, row(gate_b2))
    return out


# speedup vs baseline: 1.3126x; 1.3126x over previous
import jax
import jax.numpy as jnp
from jax.experimental import pallas as pl
from jax.experimental.pallas import tpu as pltpu

PC = (-51.2, -51.2, -5.0, 51.2, 51.2, 3.0)
E = 512
NH = 8
DH = 64
TOPK = 512
DS = 4
C = 256
HP = 64
WP = 64
T = HP * WP
N = 1220


def _pool_body(pts_ref, aw_ref, bev_ref, score_ref):
    x = pts_ref[0]
    aw = aw_ref[...]
    dims = (((1,), (1,)), ((), ()))
    for i in range(8):
        z = jnp.sum(x[:, 4 * i:4 * i + 4, :], axis=1)
        zh = jax.lax.bitcast_convert_type(
            jax.lax.bitcast_convert_type(z, jnp.int32)
            & jnp.int32(-65536), jnp.float32)
        zl = z - zh
        pooled = (jax.lax.dot_general(aw, zh, dims,
                                      preferred_element_type=jnp.float32)
                  + jax.lax.dot_general(aw, zl, dims,
                                        preferred_element_type=jnp.float32))
        bev_ref[0, WP * i:WP * (i + 1), :] = pooled
        score_ref[0, WP * i:WP * (i + 1), :] = (
            jnp.sum(pooled * pooled, axis=1, keepdims=True) * (1.0 / C))


def _ln(x, g, b):
    m = jnp.mean(x, axis=1, keepdims=True)
    v = jnp.mean((x - m) * (x - m), axis=1, keepdims=True)
    return (x - m) * jax.lax.rsqrt(v + 1e-5) * g + b


def _mm_t(x, w):
    return jax.lax.dot_general(x, w, (((1,), (1,)), ((), ())),
                               preferred_element_type=jnp.float32)


def _mm_n(x, w):
    return jnp.dot(x, w, preferred_element_type=jnp.float32)


def _fused_body(score_ref, bev_ref, inst_ref, anc_ref,
                qw_ref, kw_ref, vw_ref, bq_ref, bk_ref, bv_ref,
                aow_ref, aob_ref, ow_ref, ob_ref,
                qng_ref, qnb_ref, tng_ref, tnb_ref, ong_ref, onb_ref,
                posw1_ref, posb1_ref, posw2_ref, posb2_ref,
                ancw1_ref, ancb1_ref, ancw2_ref, ancb2_ref,
                gw1_ref, gb1_ref, gw2_ref, gb2_ref,
                out_ref):
    s = score_ref[0]
    bits = jax.lax.bitcast_convert_type(s, jnp.int32)
    lane_t = jax.lax.broadcasted_iota(jnp.int32, (1, T), 1)

    def count_ge(t):
        return jnp.sum((bits >= t).astype(jnp.int32))

    def bs_body(_, lohi):
        lo, hi = lohi
        span = hi - lo
        mid = lo + (span >> 1) + (span & 1)
        ok = count_ge(mid) >= TOPK
        return (jnp.where(ok, mid, lo), jnp.where(ok, hi, mid - 1))

    lo0 = jnp.int32(0)
    hi0 = jnp.int32(0x7F800000)
    V, _ = jax.lax.fori_loop(0, 31, bs_body, (lo0, hi0))

    n_hi = count_ge(V + 1)
    m_tie = TOPK - n_hi
    tie = bits == V

    def bs2_body(_, lohi):
        lo, hi = lohi
        mid = (lo + hi) >> 1
        cnt = jnp.sum((tie & (lane_t < mid)).astype(jnp.int32))
        ok = cnt >= m_tie
        return (jnp.where(ok, lo, mid + 1), jnp.where(ok, mid, hi))

    _, j_tie = jax.lax.fori_loop(0, 13, bs2_body,
                                 (jnp.int32(0), jnp.int32(T)))

    sel = (bits > V) | (tie & (lane_t < j_tie))
    sel_i = sel.astype(jnp.int32)

    c = sel_i
    k = 1
    while k < T:
        c = c + jnp.where(lane_t >= k, pltpu.roll(c, k, axis=1), 0)
        k *= 2
    rank = c - sel_i

    row_id = jax.lax.broadcasted_iota(jnp.int32, (TOPK, T), 0)
    onehot = (sel & (rank == row_id)).astype(jnp.float32)

    bev = bev_ref[0]
    lidar = jnp.dot(onehot, bev, preferred_element_type=jnp.float32)

    xs_n = ((lane_t % WP).astype(jnp.float32) + 0.5) * (1.0 / WP)
    ys_n = ((lane_t // WP).astype(jnp.float32) + 0.5) * (1.0 / HP)
    dims_t = (((1,), (1,)), ((), ()))
    xsel = jax.lax.dot_general(onehot, xs_n, dims_t,
                               preferred_element_type=jnp.float32)
    ysel = jax.lax.dot_general(onehot, ys_n, dims_t,
                               preferred_element_type=jnp.float32)
    w1 = posw1_ref[...]
    h1 = jnp.maximum(xsel * w1[0:1, :] + ysel * w1[1:2, :] + posb1_ref[...],
                     0.0)
    lidar = lidar + _mm_t(h1, posw2_ref[...]) + posb2_ref[...]

    kvn = _ln(lidar, tng_ref[...], tnb_ref[...])

    xy = anc_ref[0]
    xn = jnp.clip((xy[:, 0:1] - PC[0]) / (PC[3] - PC[0]), 0.0, 1.0)
    yn = jnp.clip((xy[:, 1:2] - PC[1]) / (PC[4] - PC[1]), 0.0, 1.0)
    aw1 = ancw1_ref[...]
    a1 = jnp.maximum(xn * aw1[0:1, :] + yn * aw1[1:2, :] + ancb1_ref[...],
                     0.0)
    query_input = inst_ref[0] + _mm_t(a1, ancw2_ref[...]) + ancb2_ref[...]
    query = _ln(query_input, qng_ref[...], qnb_ref[...])

    qw = qw_ref[...]
    kw = kw_ref[...]
    vw = vw_ref[...]
    aow = aow_ref[...]
    bq = bq_ref[...]
    bk = bk_ref[...]
    bv = bv_ref[...]
    scale = 1.0 / (DH ** 0.5)
    o1 = jnp.zeros((N, E), jnp.float32)
    for h in range(NH):
        sl = slice(h * DH, (h + 1) * DH)
        qh = (_mm_t(query, qw[sl, :]) + bq[h]) * scale
        kh = _mm_t(kvn, kw[sl, :]) + bk[h]
        vh = _mm_t(kvn, vw[sl, :]) + bv[h]
        logits = _mm_t(qh, kh)
        mx = jnp.max(logits, axis=1, keepdims=True)
        e = jnp.exp(logits - mx)
        p = e / jnp.sum(e, axis=1, keepdims=True)
        ctxh = _mm_n(p, vh)
        o1 = o1 + _mm_n(ctxh, aow[sl, :])
    o1 = o1 + aob_ref[...]
    o2 = _mm_t(o1, ow_ref[...]) + ob_ref[...]
    o3 = _ln(o2, ong_ref[...], onb_ref[...])

    g1 = jnp.maximum(_mm_t(query_input, gw1_ref[...]) + gb1_ref[...], 0.0)
    g2 = _mm_t(g1, gw2_ref[...]) + gb2_ref[...]
    gate = 1.0 / (1.0 + jnp.exp(-g2))
    out_ref[0] = inst_ref[0] + gate * o3


def kernel(instance_feature, pts_feats, anchor, q_proj_w, k_proj_w, v_proj_w,
           in_proj_b, attn_out_w, attn_out_b, qn_g, qn_b, tn_g, tn_b, on_g,
           on_b, out_w, out_b, pos_w1, pos_b1, pos_w2, pos_b2, anc_w1,
           anc_b1, anc_w2, anc_b2, gate_w1, gate_b1, gate_w2, gate_b2):
    B = pts_feats.shape[0]
    W = pts_feats.shape[3]

    aw = (jnp.equal(jax.lax.broadcasted_iota(jnp.int32, (WP, W), 0),
                    jax.lax.broadcasted_iota(jnp.int32, (WP, W), 1) // DS)
          .astype(jnp.float32) * (1.0 / (DS * DS)))

    bev, score = pl.pallas_call(
        _pool_body,
        grid=(B, HP // 8),
        in_specs=[
            pl.BlockSpec((1, C, 8 * DS, W), lambda b, h: (b, 0, h, 0)),
            pl.BlockSpec((WP, W), lambda b, h: (0, 0)),
        ],
        out_specs=[
            pl.BlockSpec((1, 8 * WP, C), lambda b, h: (b, h, 0)),
            pl.BlockSpec((1, 8 * WP, 1), lambda b, h: (b, h, 0)),
        ],
        out_shape=[
            jax.ShapeDtypeStruct((B, T, C), jnp.float32),
            jax.ShapeDtypeStruct((B, T, 1), jnp.float32),
        ],
    )(pts_feats, aw)

    score = score.reshape(B, 1, T)
    row = lambda x: x.reshape(1, -1)
    full = lambda *shape: pl.BlockSpec(shape, lambda b: (0,) * len(shape))
    perb = lambda *shape: pl.BlockSpec((1,) + shape,
                                       lambda b: (b,) + (0,) * len(shape))
    bq = in_proj_b[0:E].reshape(NH, 1, DH)
    bk = in_proj_b[E:2 * E].reshape(NH, 1, DH)
    bv = in_proj_b[2 * E:].reshape(NH, 1, DH)

    out = pl.pallas_call(
        _fused_body,
        grid=(B,),
        in_specs=[
            perb(1, T), perb(T, C), perb(N, E), perb(N, 2),
            full(E, E), full(E, C), full(E, C),
            full(NH, 1, DH), full(NH, 1, DH), full(NH, 1, DH),
            full(E, E), full(1, E), full(E, E), full(1, E),
            full(1, E), full(1, E), full(1, C), full(1, C),
            full(1, E), full(1, E),
            full(2, C), full(1, C), full(C, C), full(1, C),
            full(2, E), full(1, E), full(E, E), full(1, E),
            full(E, E), full(1, E), full(E, E), full(1, E),
        ],
        out_specs=pl.BlockSpec((1, N, E), lambda b: (b, 0, 0)),
        out_shape=jax.ShapeDtypeStruct((B, N, E), jnp.float32),
    )(score, bev, instance_feature, anchor[..., 0:2],
      q_proj_w, k_proj_w, v_proj_w, bq, bk, bv,
      attn_out_w.T, row(attn_out_b), out_w, row(out_b),
      row(qn_g), row(qn_b), row(tn_g), row(tn_b), row(on_g), row(on_b),
      pos_w1.T, row(pos_b1), pos_w2, row(pos_b2),
      anc_w1.T, row(anc_b1), anc_w2, row(anc_b2),
      gate_w1, row(gate_b1), gate_w2, row(gate_b2))
    return out


# speedup vs baseline: 1.6923x; 1.2892x over previous
import jax
import jax.numpy as jnp
from jax.experimental import pallas as pl
from jax.experimental.pallas import tpu as pltpu

PC = (-51.2, -51.2, -5.0, 51.2, 51.2, 3.0)
E = 512
NH = 8
DH = 64
TOPK = 512
DS = 4
C = 256
HP = 64
WP = 64
T = HP * WP
N = 1220


def _pool_body(pts_ref, aw_ref, bev_ref, score_ref):
    x = pts_ref[0]
    aw = aw_ref[...]
    dims = (((1,), (1,)), ((), ()))
    for i in range(8):
        z = jnp.sum(x[:, 4 * i:4 * i + 4, :], axis=1)
        pooled = jax.lax.dot_general(aw, z, dims,
                                     preferred_element_type=jnp.float32)
        bev_ref[0, WP * i:WP * (i + 1), :] = pooled
        score_ref[0, WP * i:WP * (i + 1), :] = (
            jnp.sum(pooled * pooled, axis=1, keepdims=True) * (1.0 / C))


def _ln(x, g, b):
    m = jnp.mean(x, axis=1, keepdims=True)
    v = jnp.mean((x - m) * (x - m), axis=1, keepdims=True)
    return (x - m) * jax.lax.rsqrt(v + 1e-5) * g + b


def _mm_t(x, w):
    return jax.lax.dot_general(x, w, (((1,), (1,)), ((), ())),
                               preferred_element_type=jnp.float32)


def _mm_n(x, w):
    return jnp.dot(x, w, preferred_element_type=jnp.float32)


def _fused_body(score_ref, bev_ref, inst_ref, anc_ref,
                qw_ref, kw_ref, vw_ref, bq_ref, bk_ref, bv_ref,
                aow_ref, aob_ref, ow_ref, ob_ref,
                qng_ref, qnb_ref, tng_ref, tnb_ref, ong_ref, onb_ref,
                posw1_ref, posb1_ref, posw2_ref, posb2_ref,
                ancw1_ref, ancb1_ref, ancw2_ref, ancb2_ref,
                gw1_ref, gb1_ref, gw2_ref, gb2_ref,
                out_ref):
    s = score_ref[0]
    bits = jax.lax.bitcast_convert_type(s, jnp.int32)
    lane_t = jax.lax.broadcasted_iota(jnp.int32, (1, T), 1)

    def count_ge(t):
        return jnp.sum((bits >= t).astype(jnp.int32))

    def bs_body(_, lohi):
        lo, hi = lohi
        span = hi - lo
        mid = lo + (span >> 1) + (span & 1)
        ok = count_ge(mid) >= TOPK
        return (jnp.where(ok, mid, lo), jnp.where(ok, hi, mid - 1))

    lo0 = jnp.int32(0)
    hi0 = jnp.int32(0x7F800000)
    V, _ = jax.lax.fori_loop(0, 31, bs_body, (lo0, hi0))

    n_hi = count_ge(V + 1)
    m_tie = TOPK - n_hi
    tie = bits == V

    def bs2_body(_, lohi):
        lo, hi = lohi
        mid = (lo + hi) >> 1
        cnt = jnp.sum((tie & (lane_t < mid)).astype(jnp.int32))
        ok = cnt >= m_tie
        return (jnp.where(ok, lo, mid + 1), jnp.where(ok, mid, hi))

    _, j_tie = jax.lax.fori_loop(0, 13, bs2_body,
                                 (jnp.int32(0), jnp.int32(T)))

    sel = (bits > V) | (tie & (lane_t < j_tie))
    sel_i = sel.astype(jnp.int32)

    c = sel_i
    k = 1
    while k < T:
        c = c + jnp.where(lane_t >= k, pltpu.roll(c, k, axis=1), 0)
        k *= 2
    rank = c - sel_i

    row_id = jax.lax.broadcasted_iota(jnp.int32, (TOPK, T), 0)
    onehot = (sel & (rank == row_id)).astype(jnp.float32)

    bev = bev_ref[0]
    lidar = jnp.dot(onehot, bev, preferred_element_type=jnp.float32)

    xs_n = ((lane_t % WP).astype(jnp.float32) + 0.5) * (1.0 / WP)
    ys_n = ((lane_t // WP).astype(jnp.float32) + 0.5) * (1.0 / HP)
    dims_t = (((1,), (1,)), ((), ()))
    xsel = jax.lax.dot_general(onehot, xs_n, dims_t,
                               preferred_element_type=jnp.float32)
    ysel = jax.lax.dot_general(onehot, ys_n, dims_t,
                               preferred_element_type=jnp.float32)
    w1 = posw1_ref[...]
    h1 = jnp.maximum(xsel * w1[0:1, :] + ysel * w1[1:2, :] + posb1_ref[...],
                     0.0)
    lidar = lidar + _mm_t(h1, posw2_ref[...]) + posb2_ref[...]

    kvn = _ln(lidar, tng_ref[...], tnb_ref[...])

    xy = anc_ref[0]
    xn = jnp.clip((xy[:, 0:1] - PC[0]) / (PC[3] - PC[0]), 0.0, 1.0)
    yn = jnp.clip((xy[:, 1:2] - PC[1]) / (PC[4] - PC[1]), 0.0, 1.0)
    aw1 = ancw1_ref[...]
    a1 = jnp.maximum(xn * aw1[0:1, :] + yn * aw1[1:2, :] + ancb1_ref[...],
                     0.0)
    query_input = inst_ref[0] + _mm_t(a1, ancw2_ref[...]) + ancb2_ref[...]
    query = _ln(query_input, qng_ref[...], qnb_ref[...])

    qw = qw_ref[...]
    kw = kw_ref[...]
    vw = vw_ref[...]
    aow = aow_ref[...]
    bq = bq_ref[...]
    bk = bk_ref[...]
    bv = bv_ref[...]
    scale = 1.0 / (DH ** 0.5)
    o1 = jnp.zeros((N, E), jnp.float32)
    for h in range(NH):
        sl = slice(h * DH, (h + 1) * DH)
        qh = (_mm_t(query, qw[sl, :]) + bq[h]) * scale
        kh = _mm_t(kvn, kw[sl, :]) + bk[h]
        vh = _mm_t(kvn, vw[sl, :]) + bv[h]
        logits = _mm_t(qh, kh)
        mx = jnp.max(logits, axis=1, keepdims=True)
        e = jnp.exp(logits - mx)
        p = e / jnp.sum(e, axis=1, keepdims=True)
        ctxh = _mm_n(p, vh)
        o1 = o1 + _mm_n(ctxh, aow[sl, :])
    o1 = o1 + aob_ref[...]
    o2 = _mm_t(o1, ow_ref[...]) + ob_ref[...]
    o3 = _ln(o2, ong_ref[...], onb_ref[...])

    g1 = jnp.maximum(_mm_t(query_input, gw1_ref[...]) + gb1_ref[...], 0.0)
    g2 = _mm_t(g1, gw2_ref[...]) + gb2_ref[...]
    gate = 1.0 / (1.0 + jnp.exp(-g2))
    out_ref[0] = inst_ref[0] + gate * o3


def kernel(instance_feature, pts_feats, anchor, q_proj_w, k_proj_w, v_proj_w,
           in_proj_b, attn_out_w, attn_out_b, qn_g, qn_b, tn_g, tn_b, on_g,
           on_b, out_w, out_b, pos_w1, pos_b1, pos_w2, pos_b2, anc_w1,
           anc_b1, anc_w2, anc_b2, gate_w1, gate_b1, gate_w2, gate_b2):
    B = pts_feats.shape[0]
    W = pts_feats.shape[3]

    aw = (jnp.equal(jax.lax.broadcasted_iota(jnp.int32, (WP, W), 0),
                    jax.lax.broadcasted_iota(jnp.int32, (WP, W), 1) // DS)
          .astype(jnp.float32) * (1.0 / (DS * DS)))

    bev, score = pl.pallas_call(
        _pool_body,
        grid=(B, HP // 8),
        in_specs=[
            pl.BlockSpec((1, C, 8 * DS, W), lambda b, h: (b, 0, h, 0)),
            pl.BlockSpec((WP, W), lambda b, h: (0, 0)),
        ],
        out_specs=[
            pl.BlockSpec((1, 8 * WP, C), lambda b, h: (b, h, 0)),
            pl.BlockSpec((1, 8 * WP, 1), lambda b, h: (b, h, 0)),
        ],
        out_shape=[
            jax.ShapeDtypeStruct((B, T, C), jnp.float32),
            jax.ShapeDtypeStruct((B, T, 1), jnp.float32),
        ],
    )(pts_feats, aw)

    score = score.reshape(B, 1, T)
    row = lambda x: x.reshape(1, -1)
    full = lambda *shape: pl.BlockSpec(shape, lambda b: (0,) * len(shape))
    perb = lambda *shape: pl.BlockSpec((1,) + shape,
                                       lambda b: (b,) + (0,) * len(shape))
    bq = in_proj_b[0:E].reshape(NH, 1, DH)
    bk = in_proj_b[E:2 * E].reshape(NH, 1, DH)
    bv = in_proj_b[2 * E:].reshape(NH, 1, DH)

    out = pl.pallas_call(
        _fused_body,
        grid=(B,),
        in_specs=[
            perb(1, T), perb(T, C), perb(N, E), perb(N, 2),
            full(E, E), full(E, C), full(E, C),
            full(NH, 1, DH), full(NH, 1, DH), full(NH, 1, DH),
            full(E, E), full(1, E), full(E, E), full(1, E),
            full(1, E), full(1, E), full(1, C), full(1, C),
            full(1, E), full(1, E),
            full(2, C), full(1, C), full(C, C), full(1, C),
            full(2, E), full(1, E), full(E, E), full(1, E),
            full(E, E), full(1, E), full(E, E), full(1, E),
        ],
        out_specs=pl.BlockSpec((1, N, E), lambda b: (b, 0, 0)),
        out_shape=jax.ShapeDtypeStruct((B, N, E), jnp.float32),
    )(score, bev, instance_feature, anchor[..., 0:2],
      q_proj_w, k_proj_w, v_proj_w, bq, bk, bv,
      attn_out_w.T, row(attn_out_b), out_w, row(out_b),
      row(qn_g), row(qn_b), row(tn_g), row(tn_b), row(on_g), row(on_b),
      pos_w1.T, row(pos_b1), pos_w2, row(pos_b2),
      anc_w1.T, row(anc_b1), anc_w2, row(anc_b2),
      gate_w1, row(gate_b1), gate_w2, row(gate_b2))
    return out
